# baseline (speedup 1.0000x reference)
import functools

import jax
import jax.numpy as jnp
from jax import lax
from jax.experimental import pallas as pl
from jax.experimental.pallas import tpu as pltpu

EPS = 1e-5
GRP = 2


def _batch_block(b, max_tb):
    tb = min(b, max_tb)
    while b % tb:
        tb -= 1
    return tb


def _conv_pool_kernel(xm_ref, wb_ref, cb_ref, feat_ref, *, c):
    tb, h, k = xm_ref.shape
    f = feat_ref.shape[-1]
    w = k // c
    wg = w // GRP
    sl = wg + 2
    ng = wg * f
    xm = xm_ref[...]
    zrow = jnp.zeros((tb, 1, k), jnp.float32)
    dn = jnp.concatenate([zrow, xm[:, :h - 1, :]], axis=1)
    up = jnp.concatenate([xm[:, 1:, :], zrow], axis=1)

    def grp_lanes(v, lo, hi):
        return v.reshape(tb, h, c, w)[..., lo:hi].reshape(tb, h, c * (hi - lo))

    lhs = []
    for g in range(GRP):
        lo = max(0, g * wg - 1)
        hi = min(w, (g + 1) * wg + 1)
        lhs.append(jnp.concatenate(
            [grp_lanes(dn, lo, hi), grp_lanes(xm, lo, hi),
             grp_lanes(up, lo, hi)], axis=-1).reshape(tb * h, -1))

    imgs = max(1, min(tb, 256 // h))
    while tb % imgs:
        imgs -= 1
    parts = []
    for b0 in range(0, tb, imgs):
        row0, row1 = b0 * h, (b0 + imgs) * h
        ssum = None
        for g in range(GRP):
            a = jnp.dot(lhs[g][row0:row1], wb_ref[g],
                        preferred_element_type=jnp.float32)
            yc = jnp.maximum(a + cb_ref[...], 0.0)
            sg = jnp.sum(yc.reshape(imgs, h, wg, f), axis=(1, 2))
            ssum = sg if ssum is None else ssum + sg
        parts.append(ssum)
    feat_ref[...] = jnp.concatenate(parts, axis=0) * (1.0 / (h * w))


def _head_kernel(feat_ref, w1_ref, b1_ref, g1_ref, be1_ref,
                 w2_ref, b2_ref, g2_ref, o_ref):
    feat = feat_ref[...]
    h = jnp.dot(feat, w1_ref[...],
                preferred_element_type=jnp.float32) + b1_ref[...]
    mu = jnp.mean(h, axis=0, keepdims=True)
    var = jnp.mean((h - mu) ** 2, axis=0, keepdims=True)
    h = (h - mu) * lax.rsqrt(var + EPS) * g1_ref[...] + be1_ref[...]
    h = jnp.maximum(h, 0.0)
    z = jnp.dot(h, w2_ref[...],
                preferred_element_type=jnp.float32) + b2_ref[...]
    mu2 = jnp.mean(z, axis=0, keepdims=True)
    var2 = jnp.mean((z - mu2) ** 2, axis=0, keepdims=True)
    o_ref[...] = (z - mu2) * lax.rsqrt(var2 + EPS) * g2_ref[...]


def kernel(x, conv_w, conv_b, w1, b1, g1, be1, w2, b2, g2):
    B, C, H, W = x.shape
    F = conv_w.shape[-1]
    hidden = w1.shape[-1]
    out_dim = w2.shape[-1]
    K = C * W
    WG = W // GRP
    NG = WG * F

    xm = jnp.transpose(x, (0, 2, 1, 3)).reshape(B, H, K)

    cw = conv_w.astype(jnp.float32)
    groups = []
    for g in range(GRP):
        lo = max(0, g * WG - 1)
        hi = min(W, (g + 1) * WG + 1)
        slabs = []
        for di in range(3):
            acc = jnp.zeros((C, hi - lo, WG, F), jnp.float32)
            for dj in range(3):
                eye = jnp.eye(W, W, k=1 - dj,
                              dtype=jnp.float32)[lo:hi, g * WG:(g + 1) * WG]
                acc = acc + jnp.einsum('Jj,cf->cJjf', eye, cw[3 * di + dj])
            slabs.append(acc.reshape(C * (hi - lo), NG))
        groups.append(jnp.concatenate(slabs, axis=0))
    wb = jnp.stack(groups)

    cb_big = jnp.tile(conv_b.astype(jnp.float32), (1, WG))

    TB = _batch_block(B, 64)
    nblk = B // TB
    conv_flops = 2 * B * H * 3 * K * W * F
    conv_bytes = 4 * (xm.size + wb.size + B * F)

    feats = pl.pallas_call(
        functools.partial(_conv_pool_kernel, c=C),
        out_shape=jax.ShapeDtypeStruct((B, F), jnp.float32),
        grid=(nblk,),
        in_specs=[
            pl.BlockSpec((TB, H, K), lambda i: (i, 0, 0)),
            pl.BlockSpec(wb.shape, lambda i: (0, 0, 0)),
            pl.BlockSpec((1, NG), lambda i: (0, 0)),
        ],
        out_specs=pl.BlockSpec((TB, F), lambda i: (i, 0)),
        compiler_params=pltpu.CompilerParams(
            dimension_semantics=("parallel",)),
        cost_estimate=pl.CostEstimate(flops=conv_flops, transcendentals=0,
                                      bytes_accessed=conv_bytes),
    )(xm, wb, cb_big)

    head_flops = 2 * B * F * hidden + 2 * B * hidden * out_dim
    head_bytes = 4 * (feats.size + w1.size + w2.size
                      + 3 * hidden + 3 * out_dim + B * out_dim)
    out = pl.pallas_call(
        _head_kernel,
        out_shape=jax.ShapeDtypeStruct((B, out_dim), jnp.float32),
        cost_estimate=pl.CostEstimate(flops=head_flops,
                                      transcendentals=hidden + out_dim,
                                      bytes_accessed=head_bytes),
    )(feats, w1, b1, g1, be1, w2, b2, g2)

    return out
